# Initial kernel scaffold; baseline (speedup 1.0000x reference)
import jax
import jax.numpy as jnp
from jax.experimental import pallas as pl


def kernel(test_features, centroids):
    raise NotImplementedError("write your pallas kernel here")


# speedup vs baseline: 1.4069x; 1.4069x over previous
import jax
import jax.numpy as jnp
from jax.experimental import pallas as pl

_BN = 1024


def _assign_kernel(x_ref, c_ref, out_ref):
    x = x_ref[...]
    c = c_ref[...]
    x_sq = jnp.sum(x * x, axis=1, keepdims=True)
    c_sq = jnp.sum(c * c, axis=0, keepdims=True)
    cross = jax.lax.dot_general(
        x, c, (((1,), (0,)), ((), ())),
        preferred_element_type=jnp.float32)
    scores = x_sq - 2.0 * cross + c_sq
    out_ref[...] = jnp.argmin(scores, axis=1).astype(jnp.int32)


def kernel(test_features, centroids):
    n, d = test_features.shape
    k = centroids.shape[1]
    return pl.pallas_call(
        _assign_kernel,
        grid=(n // _BN,),
        in_specs=[
            pl.BlockSpec((_BN, d), lambda i: (i, 0)),
            pl.BlockSpec((d, k), lambda i: (0, 0)),
        ],
        out_specs=pl.BlockSpec((_BN,), lambda i: (i,)),
        out_shape=jax.ShapeDtypeStruct((n,), jnp.int32),
    )(test_features, centroids)
